# baseline (speedup 1.0000x reference)
import jax
import jax.numpy as jnp
from jax import lax
from jax.experimental import pallas as pl
from jax.experimental.pallas import tpu as pltpu
from jax.experimental.pallas import tpu_sc as plsc

NUM_CORES = 2
NUM_SUBCORES = 16
NUM_WORKERS = NUM_CORES * NUM_SUBCORES
CHUNK_B = 32
BLK_B = 32


def _gather_kernel(table_hbm, ids_hbm, out_hbm, idx2_v, idx_v, rows_v, sem):
    n_batch, seq = ids_hbm.shape
    dim = table_hbm.shape[1]
    half_rows = out_hbm.shape[0]
    b_per_w = n_batch // NUM_WORKERS
    wid = lax.axis_index("s") * NUM_CORES + lax.axis_index("c")
    b0w = wid * b_per_w
    n_ids = CHUNK_B * seq
    lane = lax.iota(jnp.int32, 16)

    @pl.loop(0, b_per_w, step=CHUNK_B)
    def _(bo):
        b0 = b0w + bo
        pltpu.sync_copy(ids_hbm.at[pl.ds(b0, CHUNK_B)], idx2_v)
        for j in range(n_ids // 16):
            flat = lane + (16 * j)
            vals = plsc.load_gather(idx2_v, [flat // seq, flat % seq])
            idx_v[pl.ds(16 * j, 16)] = vals
        pltpu.async_copy(table_hbm.at[idx_v], rows_v, sem).wait()
        t0 = b0 * seq
        row0 = lax.rem(t0, half_rows)
        col0 = lax.div(t0, half_rows) * dim
        pltpu.sync_copy(
            rows_v, out_hbm.at[pl.ds(row0, n_ids), pl.ds(col0, dim)]
        )


def _relayout_kernel(src_ref, out_ref):
    x = src_ref[...]
    h = pl.program_id(1)
    dim = out_ref.shape[-1]
    half = jnp.where(h == 0, x[:, :dim], x[:, dim:])
    out_ref[...] = half.reshape(out_ref.shape)


def kernel(token_ids, embedding_table):
    batch, seq = token_ids.shape
    dim = embedding_table.shape[1]
    ids = token_ids.astype(jnp.int32)
    half_rows = batch * seq // 2

    mesh = plsc.VectorSubcoreMesh(core_axis_name="c", subcore_axis_name="s")
    gather = pl.kernel(
        _gather_kernel,
        mesh=mesh,
        out_type=jax.ShapeDtypeStruct((half_rows, 2 * dim), embedding_table.dtype),
        scratch_types=[
            pltpu.VMEM((CHUNK_B, seq), jnp.int32),
            pltpu.VMEM((CHUNK_B * seq,), jnp.int32),
            pltpu.VMEM((CHUNK_B * seq, dim), jnp.float32),
            pltpu.SemaphoreType.DMA,
        ],
        compiler_params=pltpu.CompilerParams(
            use_tc_tiling_on_sc=False, needs_layout_passes=False
        ),
    )
    inter = gather(embedding_table, ids)

    half_blocks = batch // (2 * BLK_B)
    relayout = pl.pallas_call(
        _relayout_kernel,
        out_shape=jax.ShapeDtypeStruct((batch, seq, dim), embedding_table.dtype),
        grid=(half_blocks, 2),
        in_specs=[pl.BlockSpec((BLK_B * seq, 2 * dim), lambda i, h: (i, 0))],
        out_specs=pl.BlockSpec(
            (BLK_B, seq, dim), lambda i, h: (h * half_blocks + i, 0, 0)
        ),
    )
    return relayout(inter)
